# baseline (speedup 1.0000x reference)
import functools
import math

import jax
import jax.numpy as jnp
from jax import lax
from jax.experimental import pallas as pl
from jax.experimental.pallas import tpu as pltpu
from jax.experimental.pallas import tpu_sc as plsc

_INTERPRET = False

_NC = 2
_NS = 16
_L = 16
_NW = _NC * _NS


def _pad_to(a, axis, m):
    s = a.shape[axis]
    p = (-s) % m
    if p == 0:
        return a
    pads = [(0, 0)] * a.ndim
    pads[axis] = (0, p)
    return jnp.pad(a, pads)


def _ceil_to(x, m):
    return -(-x // m) * m


def _mm_kernel(a_ref, b_ref, o_ref, *, kk, nn, bk, bn, nk):
    k = pl.program_id(2)
    j = pl.program_id(1)

    @pl.when(k == 0)
    def _():
        o_ref[...] = jnp.zeros_like(o_ref)

    a = a_ref[...]
    kcol = lax.broadcasted_iota(jnp.int32, a.shape, 1) + k * bk
    a = jnp.where(kcol < kk, a, 0.0)
    o_ref[...] += lax.dot_general(
        a, b_ref[...], (((1,), (0,)), ((), ())),
        preferred_element_type=jnp.float32,
        precision=lax.Precision.DEFAULT)

    @pl.when(k == nk - 1)
    def _():
        ocol = lax.broadcasted_iota(jnp.int32, o_ref.shape, 1) + j * bn
        o_ref[...] = jnp.where(ocol < nn, o_ref[...], 0.0)


def _pick_bn(npad):
    m = npad // 128
    best = 1
    for nj in range(1, m + 1):
        if m % nj == 0 and (m // nj) * 128 <= 2048:
            best = m // nj
            break
    return best * 128


def _mm_padded(a, b, npad, bm=1024, bk=512):
    M, K = a.shape
    _, N = b.shape
    bk = min(bk, _ceil_to(K, 128))
    bn = _pick_bn(npad)
    nk = -(-K // bk)
    grid = (M // bm, npad // bn, nk)
    return pl.pallas_call(
        functools.partial(_mm_kernel, kk=K, nn=N, bk=bk, bn=bn, nk=nk),
        grid=grid,
        in_specs=[pl.BlockSpec((bm, bk), lambda i, j, k: (i, k)),
                  pl.BlockSpec((bk, bn), lambda i, j, k: (k, j))],
        out_specs=pl.BlockSpec((bm, bn), lambda i, j, k: (i, j)),
        out_shape=jax.ShapeDtypeStruct((M, npad), jnp.float32),
        compiler_params=pltpu.CompilerParams(
            dimension_semantics=("parallel", "parallel", "arbitrary")),
        interpret=_INTERPRET,
    )(a, b)


def _relayout_kernel(x_ref, o_ref):
    o_ref[...] = x_ref[...]


def _chunk_rows(v):
    npad, dp = v.shape
    nc = dp // 128
    bm = 512
    ni = npad // bm
    out = pl.pallas_call(
        _relayout_kernel,
        grid=(ni, nc),
        in_specs=[pl.BlockSpec((bm, 128), lambda i, j: (i, j))],
        out_specs=pl.BlockSpec((bm, 128), lambda i, j: (j * ni + i, 0)),
        out_shape=jax.ShapeDtypeStruct((nc * npad, 128), jnp.float32),
        compiler_params=pltpu.CompilerParams(
            dimension_semantics=("parallel", "parallel")),
        interpret=_INTERPRET,
    )(v)
    return out


def _ln_relu_kernel(x_ref, g_ref, b_ref, o_ref, *, d):
    x = x_ref[...]
    mu = jnp.sum(x, axis=-1, keepdims=True) / d
    xc = x - mu
    var = jnp.sum(jnp.where(lax.broadcasted_iota(jnp.int32, x.shape, 1) < d,
                            xc * xc, 0.0), axis=-1, keepdims=True) / d
    y = g_ref[...] * xc * lax.rsqrt(var + 1e-5) + b_ref[...]
    o_ref[...] = jnp.maximum(y, 0.0)


def _ln_relu(x, g, b, bm=256):
    R, D = x.shape
    d = g.shape[0]
    xp = _pad_to(_pad_to(x, 0, bm), 1, 128)
    Rp, Dp = xp.shape
    gp = _pad_to(g[None, :], 1, Dp)[:, :Dp]
    bp = _pad_to(b[None, :], 1, Dp)[:, :Dp]
    out = pl.pallas_call(
        functools.partial(_ln_relu_kernel, d=d),
        grid=(Rp // bm,),
        in_specs=[pl.BlockSpec((bm, Dp), lambda i: (i, 0)),
                  pl.BlockSpec((1, Dp), lambda i: (0, 0)),
                  pl.BlockSpec((1, Dp), lambda i: (0, 0))],
        out_specs=pl.BlockSpec((bm, Dp), lambda i: (i, 0)),
        out_shape=jax.ShapeDtypeStruct((Rp, Dp), jnp.float32),
        interpret=_INTERPRET,
    )(xp, gp, bp)
    return out[:R]


def _sigmoid_kernel(x_ref, o_ref):
    o_ref[...] = jax.nn.sigmoid(x_ref[...])


def _sigmoid(x, bm=512):
    R, D = x.shape
    xp = _pad_to(_pad_to(x, 0, bm), 1, 128)
    Rp, Dp = xp.shape
    out = pl.pallas_call(
        _sigmoid_kernel,
        grid=(Rp // bm,),
        in_specs=[pl.BlockSpec((bm, Dp), lambda i: (i, 0))],
        out_specs=pl.BlockSpec((bm, Dp), lambda i: (i, 0)),
        out_shape=jax.ShapeDtypeStruct((Rp, Dp), jnp.float32),
        interpret=_INTERPRET,
    )(xp)
    return out[:R, :D]


def _sc_mesh():
    return plsc.VectorSubcoreMesh(core_axis_name="c", subcore_axis_name="s")


def _hsum16(v):
    s = v[0]
    for l in range(1, _L):
        s = s + v[l]
    return s


def _hmax16(v):
    s = v[0]
    for l in range(1, _L):
        s = jnp.maximum(s, v[l])
    return s


def _vec16(scalars):
    lanes = lax.iota(jnp.int32, _L)
    v = jnp.full((_L,), scalars[0], jnp.float32)
    for l in range(1, _L):
        v = jnp.where(lanes == l, jnp.full((_L,), scalars[l], jnp.float32), v)
    return v


def _alpha_call(qt, kt, ep, srcp, dstp, scale, npad):
    Epad, Dp = ep.shape
    nc = Dp // 128
    EPW = Epad // _NW
    G = 64
    ng = EPW // G
    T = nc * ng

    @functools.partial(
        pl.kernel, mesh=_sc_mesh(),
        out_type=jax.ShapeDtypeStruct((Epad,), jnp.float32),
        scratch_types=[
            pltpu.VMEM((EPW,), jnp.int32),
            pltpu.VMEM((EPW,), jnp.int32),
            [pltpu.VMEM((G,), jnp.int32) for _ in range(2)],
            [pltpu.VMEM((G,), jnp.int32) for _ in range(2)],
            [pltpu.VMEM((G, 128), jnp.float32) for _ in range(2)],
            [pltpu.VMEM((G, 128), jnp.float32) for _ in range(2)],
            [pltpu.VMEM((G, 128), jnp.float32) for _ in range(2)],
            pltpu.VMEM((EPW * _L,), jnp.float32),
            pltpu.VMEM((EPW,), jnp.float32),
            [pltpu.SemaphoreType.DMA for _ in range(2)],
        ])
    def k_alpha(q_h, k_h, e_h, src_h, dst_h, al_h,
                src_v, dst_v, qg, kg, qb, kb, eb, pb, ab, sem):
        wid = lax.axis_index("s") * _NC + lax.axis_index("c")
        base = wid * EPW
        pltpu.sync_copy(src_h.at[pl.ds(base, EPW)], src_v)
        pltpu.sync_copy(dst_h.at[pl.ds(base, EPW)], dst_v)

        def zpb(i, c):
            pb[pl.ds(i * _L, _L)] = jnp.zeros((_L,), jnp.float32)
            return c

        lax.fori_loop(0, EPW, zpb, 0)

        def fire(it, p):
            ci = it // ng
            g = it - ci * ng
            gb = g * G
            for t in range(G // _L):
                sl_s = pl.ds(gb + t * _L, _L)
                sl_d = pl.ds(t * _L, _L)
                qg[p][sl_d] = dst_v[sl_s] + ci * npad
                kg[p][sl_d] = src_v[sl_s] + ci * npad
            pltpu.async_copy(q_h.at[qg[p]], qb[p], sem[p])
            pltpu.async_copy(k_h.at[kg[p]], kb[p], sem[p])
            pltpu.async_copy(
                e_h.at[pl.ds(base + gb, G), pl.ds(ci * 128, 128)],
                eb[p], sem[p])

        def drain(p):
            pltpu.make_async_copy(q_h.at[qg[p]], qb[p], sem[p]).wait()
            pltpu.make_async_copy(k_h.at[kg[p]], kb[p], sem[p]).wait()
            pltpu.make_async_copy(
                e_h.at[pl.ds(0, G), pl.ds(0, 128)], eb[p], sem[p]).wait()

        def compute(it, p):
            ci = it // ng
            g = it - ci * ng
            gb = g * G
            for i in range(G):
                acc = qb[p][i, pl.ds(0, _L)] * (
                    kb[p][i, pl.ds(0, _L)] + eb[p][i, pl.ds(0, _L)])
                for j8 in range(1, 128 // _L):
                    sl = pl.ds(j8 * _L, _L)
                    acc = acc + qb[p][i, sl] * (kb[p][i, sl] + eb[p][i, sl])
                po = pl.ds((gb + i) * _L, _L)
                pb[po] = pb[po] + acc

        fire(0, 0)

        def pair(j, c):
            it0 = j * 2

            @pl.when(it0 + 1 < T)
            def _():
                fire(it0 + 1, 1)
            drain(0)
            compute(it0, 0)

            @pl.when(it0 + 2 < T)
            def _():
                fire(it0 + 2, 0)

            @pl.when(it0 + 1 < T)
            def _():
                drain(1)
                compute(it0 + 1, 1)
            return c

        lax.fori_loop(0, (T + 1) // 2, pair, 0)

        def batch(b, carry):
            sums = [_hsum16(pb[pl.ds((b * _L + l) * _L, _L)])
                    for l in range(_L)]
            ab[pl.ds(b * _L, _L)] = _vec16(sums) * scale
            return carry

        lax.fori_loop(0, EPW // _L, batch, 0)
        pltpu.sync_copy(ab, al_h.at[pl.ds(base, EPW)])

    return k_alpha(qt, kt, ep, srcp, dstp)


def _softmax_call(alpha, dstp, n_edges, npad):
    Epad = alpha.shape[0]
    EPT = Epad // _NS
    nch = EPT // _L
    slab = npad // _NS

    slabn = npad // _NS

    @functools.partial(
        pl.kernel, mesh=_sc_mesh(),
        out_type=jax.ShapeDtypeStruct((Epad,), jnp.float32),
        scratch_types=[
            pltpu.VMEM((EPT,), jnp.float32),
            pltpu.VMEM((EPT,), jnp.int32),
            pltpu.VMEM((EPT,), jnp.float32),
            pltpu.VMEM((EPT,), jnp.float32),
            pltpu.VMEM((npad,), jnp.float32),
            pltpu.VMEM((_L,), jnp.float32),
            pltpu.VMEM((_NS * _L,), jnp.float32),
            pltpu.VMEM((slabn,), jnp.float32),
            pltpu.VMEM((slabn,), jnp.float32),
            pltpu.VMEM_SHARED((_NS * _L,), jnp.float32),
            pltpu.VMEM_SHARED((_NS * npad,), jnp.float32),
            pltpu.VMEM_SHARED((npad,), jnp.float32),
            pltpu.SemaphoreType.DMA,
        ])
    def k_soft(a_h, dst_h, al_h,
               av, dv, exv, alv, sfl, mrow, mall, acc, tmp,
               mx_sh, s_all, s_comb, sem):
        cid = lax.axis_index("c")
        tid = lax.axis_index("s")

        @pl.when(cid == 0)
        def _():
            base = tid * EPT
            pltpu.sync_copy(a_h.at[pl.ds(base, EPT)], av)
            pltpu.sync_copy(dst_h.at[pl.ds(base, EPT)], dv)
            lanes = lax.iota(jnp.int32, _L)

            def mstep(i, m):
                idx = base + i * _L + lanes
                a = av[pl.ds(i * _L, _L)]
                return jnp.maximum(m, jnp.where(idx < n_edges, a, -1e30))

            m16 = lax.fori_loop(0, nch, mstep,
                                jnp.full((_L,), -1e30, jnp.float32))
            mrow[...] = jnp.full((_L,), _hmax16(m16), jnp.float32)
            pltpu.sync_copy(mrow, mx_sh.at[pl.ds(tid * _L, _L)])
            plsc.subcore_barrier()
            pltpu.sync_copy(mx_sh, mall)

            def mstep2(i, m):
                return jnp.maximum(m, mall[pl.ds(i * _L, _L)])

            mg = _hmax16(lax.fori_loop(0, _NS, mstep2,
                                       jnp.full((_L,), -1e30, jnp.float32)))

            def estep(i, c):
                sl = pl.ds(i * _L, _L)
                idx = base + i * _L + lanes
                exv[sl] = jnp.where(idx < n_edges,
                                    jnp.exp(av[sl] - mg), 0.0)
                return c

            lax.fori_loop(0, nch, estep, 0)

            def zstep(i, c):
                sfl[pl.ds(i * _L, _L)] = jnp.zeros((_L,), jnp.float32)
                return c

            lax.fori_loop(0, npad // _L, zstep, 0)

            def sstep(i, c):
                d16 = dv[pl.ds(i * _L, _L)]
                ex16 = exv[pl.ds(i * _L, _L)]
                for l in range(_L):
                    d = d16[l]
                    off = (d // _L) * _L
                    lp = d - off
                    cur = sfl[pl.ds(off, _L)]
                    upd = jnp.where(lanes == lp,
                                    jnp.full((_L,), ex16[l], jnp.float32),
                                    jnp.zeros((_L,), jnp.float32))
                    sfl[pl.ds(off, _L)] = cur + upd
                return c

            lax.fori_loop(0, nch, sstep, 0)

            pltpu.sync_copy(sfl, s_all.at[pl.ds(tid * npad, npad)])
            plsc.subcore_barrier()

            def zacc(i, c):
                acc[pl.ds(i * _L, _L)] = jnp.zeros((_L,), jnp.float32)
                return c

            lax.fori_loop(0, slabn // _L, zacc, 0)

            def jsum(j, c):
                pltpu.sync_copy(
                    s_all.at[pl.ds(j * npad + tid * slabn, slabn)], tmp)

                def astep(i, c2):
                    sl = pl.ds(i * _L, _L)
                    acc[sl] = acc[sl] + tmp[sl]
                    return c2

                lax.fori_loop(0, slabn // _L, astep, 0)
                return c

            lax.fori_loop(0, _NS, jsum, 0)
            pltpu.sync_copy(acc, s_comb.at[pl.ds(tid * slabn, slabn)])
            plsc.subcore_barrier()
            pltpu.sync_copy(s_comb, sfl)

            def nstep(i, c):
                sl = pl.ds(i * _L, _L)
                d16 = dv[sl]
                svals = []
                for l in range(_L):
                    d = d16[l]
                    off = (d // _L) * _L
                    lp = d - off
                    slot = sfl[pl.ds(off, _L)]
                    svals.append(_hsum16(jnp.where(
                        lanes == lp, slot, jnp.zeros((_L,), jnp.float32))))
                s16 = _vec16(svals)
                alv[sl] = exv[sl] / (s16 + 1e-16)
                return c

            lax.fori_loop(0, nch, nstep, 0)
            pltpu.sync_copy(alv, al_h.at[pl.ds(base, EPT)])

    return k_soft(alpha, dstp)


def _agg_call(vt2, ep, srcp, dstp, al, npad):
    Epad, Dp = ep.shape
    nc = Dp // 128
    EPT = Epad // _NS
    G = 64
    ng = EPT // G
    ncl = -(-nc // _NC)
    slab = npad // _NS

    @functools.partial(
        pl.kernel, mesh=_sc_mesh(),
        out_type=jax.ShapeDtypeStruct((npad, Dp), jnp.float32),
        scratch_types=[
            pltpu.VMEM((EPT,), jnp.int32),
            pltpu.VMEM((EPT,), jnp.int32),
            pltpu.VMEM((EPT,), jnp.float32),
            [pltpu.VMEM((G,), jnp.int32) for _ in range(2)],
            [pltpu.VMEM((G,), jnp.int32) for _ in range(2)],
            [pltpu.VMEM((G, 128), jnp.float32) for _ in range(2)],
            [pltpu.VMEM((G, 128), jnp.float32) for _ in range(2)],
            pltpu.VMEM((G, 128), jnp.float32),
            pltpu.VMEM((64, 128), jnp.float32),
            pltpu.VMEM_SHARED((npad, 128), jnp.float32),
            [pltpu.SemaphoreType.DMA for _ in range(2)],
        ])
    def k_agg(v_h, e_h, src_h, dst_h, al_h, out_h,
              src_v, dst_v, al_v, sg, dg, vb, eb, ob, zb, acc_sh, sem):
        cid = lax.axis_index("c")
        tid = lax.axis_index("s")
        base = tid * EPT
        pltpu.sync_copy(src_h.at[pl.ds(base, EPT)], src_v)
        pltpu.sync_copy(dst_h.at[pl.ds(base, EPT)], dst_v)
        pltpu.sync_copy(al_h.at[pl.ds(base, EPT)], al_v)

        def zrow(i, c):
            zb[i, :] = jnp.zeros((128,), jnp.float32)
            return c

        lax.fori_loop(0, 64, zrow, 0)

        def chunk(j, carry):
            ci = j * _NC + cid

            @pl.when(ci < nc)
            def _():
                def zsh(z, c):
                    pltpu.sync_copy(
                        zb, acc_sh.at[pl.ds(tid * slab + z * 64, 64)])
                    return c

                lax.fori_loop(0, slab // 64, zsh, 0)
                plsc.subcore_barrier()

                def fire(g, p):
                    gb = g * G
                    for t in range(G // _L):
                        sl_s = pl.ds(gb + t * _L, _L)
                        sl_d = pl.ds(t * _L, _L)
                        sg[p][sl_d] = src_v[sl_s] + ci * npad
                        dg[p][sl_d] = dst_v[sl_s]
                    pltpu.async_copy(v_h.at[sg[p]], vb[p], sem[p])
                    pltpu.async_copy(
                        e_h.at[pl.ds(base + gb, G), pl.ds(ci * 128, 128)],
                        eb[p], sem[p])

                def drain(p):
                    pltpu.make_async_copy(
                        v_h.at[sg[p]], vb[p], sem[p]).wait()
                    pltpu.make_async_copy(
                        e_h.at[pl.ds(0, G), pl.ds(0, 128)],
                        eb[p], sem[p]).wait()

                def work(g, p):
                    gb = g * G
                    for t in range(G // _L):
                        al16 = al_v[pl.ds(gb + t * _L, _L)]
                        for l in range(_L):
                            i = t * _L + l
                            a_sc = al16[l]
                            for j8 in range(128 // _L):
                                sl = pl.ds(j8 * _L, _L)
                                ob[i, sl] = (vb[p][i, sl]
                                             + eb[p][i, sl]) * a_sc
                    pltpu.sync_copy(ob, acc_sh.at[dg[p]], add=True)

                fire(0, 0)

                def gpair(q, c):
                    g0 = q * 2
                    fire(g0 + 1, 1)
                    drain(0)
                    work(g0, 0)

                    @pl.when(g0 + 2 < ng)
                    def _():
                        fire(g0 + 2, 0)
                    drain(1)
                    work(g0 + 1, 1)
                    return c

                lax.fori_loop(0, ng // 2, gpair, 0)
                plsc.subcore_barrier()
                pltpu.sync_copy(
                    acc_sh.at[pl.ds(tid * slab, slab)],
                    out_h.at[pl.ds(tid * slab, slab), pl.ds(ci * 128, 128)])

            return carry

        lax.fori_loop(0, ncl, chunk, 0)

    return k_agg(vt2, ep, srcp, dstp, al)


def _edge_phase_sc(qp, kp, vp, ep, srcp, dstp, n_edges, d_out):
    npad, dp = qp.shape
    nc = dp // 128
    scale = 1.0 / math.sqrt(float(d_out))
    alpha = _alpha_call(_chunk_rows(qp), _chunk_rows(kp), ep, srcp, dstp,
                        scale, npad)
    al = _softmax_call(alpha, dstp, n_edges, npad)
    vt2 = _chunk_rows(vp)
    agg = _agg_call(vt2, ep, srcp, dstp, al, npad)
    return agg, al


_NPAD = 5120
_EPAD = 10240


def _tconv(xp, eap, srcp, dstp, p, d_out, n_edges):
    dp = _ceil_to(d_out, 128)
    q = _mm_padded(xp, p['Wq'], dp, bm=1024)
    k = _mm_padded(xp, p['Wk'], dp, bm=1024)
    v = _mm_padded(xp, p['Wv'], dp, bm=1024)
    e = _mm_padded(eap, p['We'], dp, bm=1024)
    agg, al = _edge_phase_sc(q, k, v, e, srcp, dstp, n_edges, d_out)
    s = _mm_padded(xp, p['Ws'], dp, bm=1024)
    return agg + s, al


def kernel(x, edge_index, edge_attr, params):
    n, _ = x.shape
    e_cnt = edge_attr.shape[0]
    src = _pad_to(edge_index[0], 0, _EPAD)
    dst = _pad_to(edge_index[1], 0, _EPAD)
    xp = _pad_to(x, 0, _NPAD)
    ea0 = _pad_to(edge_attr, 0, _EPAD)
    ea = ea0
    dims = [3400, 2800, 2200, 1600, 1000]
    for i, d_out in enumerate(dims):
        p = params['conv%d' % (i + 1)]
        xp, al = _tconv(xp, ea, src, dst, p, d_out, e_cnt)
        if i < 4:
            nrm = params['norm%d' % (i + 1)]
            xp = _ln_relu(xp, nrm['g'], nrm['b'])[:, :_ceil_to(d_out, 128)]
            nrm1 = params['norm%d_1' % (i + 1)]
            ea = _ln_relu(jnp.concatenate([ea0, al[:, None]], axis=1),
                          nrm1['g'], nrm1['b'])[:, :24]
    return _sigmoid(xp[:n, :1000])


# speedup vs baseline: 1.2103x; 1.2103x over previous
import functools
import math

import jax
import jax.numpy as jnp
from jax import lax
from jax.experimental import pallas as pl
from jax.experimental.pallas import tpu as pltpu
from jax.experimental.pallas import tpu_sc as plsc

_INTERPRET = False

_NC = 2
_NS = 16
_L = 16
_NW = _NC * _NS


def _pad_to(a, axis, m):
    s = a.shape[axis]
    p = (-s) % m
    if p == 0:
        return a
    pads = [(0, 0)] * a.ndim
    pads[axis] = (0, p)
    return jnp.pad(a, pads)


def _ceil_to(x, m):
    return -(-x // m) * m


def _mm_kernel(a_ref, b_ref, o_ref, *, kk, nn, bk, bn, nk):
    k = pl.program_id(2)
    j = pl.program_id(1)

    @pl.when(k == 0)
    def _():
        o_ref[...] = jnp.zeros_like(o_ref)

    a = a_ref[...]
    kcol = lax.broadcasted_iota(jnp.int32, a.shape, 1) + k * bk
    a = jnp.where(kcol < kk, a, 0.0)
    o_ref[...] += lax.dot_general(
        a, b_ref[...], (((1,), (0,)), ((), ())),
        preferred_element_type=jnp.float32,
        precision=lax.Precision.DEFAULT)

    @pl.when(k == nk - 1)
    def _():
        ocol = lax.broadcasted_iota(jnp.int32, o_ref.shape, 1) + j * bn
        o_ref[...] = jnp.where(ocol < nn, o_ref[...], 0.0)


def _pick_bn(npad):
    m = npad // 128
    best = 1
    for nj in range(1, m + 1):
        if m % nj == 0 and (m // nj) * 128 <= 2048:
            best = m // nj
            break
    return best * 128


def _mm_padded(a, b, npad, bm=1024, bk=512):
    M, K = a.shape
    _, N = b.shape
    bk = min(bk, _ceil_to(K, 128))
    bn = _pick_bn(npad)
    nk = -(-K // bk)
    grid = (M // bm, npad // bn, nk)
    return pl.pallas_call(
        functools.partial(_mm_kernel, kk=K, nn=N, bk=bk, bn=bn, nk=nk),
        grid=grid,
        in_specs=[pl.BlockSpec((bm, bk), lambda i, j, k: (i, k)),
                  pl.BlockSpec((bk, bn), lambda i, j, k: (k, j))],
        out_specs=pl.BlockSpec((bm, bn), lambda i, j, k: (i, j)),
        out_shape=jax.ShapeDtypeStruct((M, npad), jnp.float32),
        compiler_params=pltpu.CompilerParams(
            dimension_semantics=("parallel", "parallel", "arbitrary")),
        interpret=_INTERPRET,
    )(a, b)


def _relayout_kernel(x_ref, o_ref):
    o_ref[...] = x_ref[...]


def _chunk_rows(v):
    npad, dp = v.shape
    nc = dp // 128
    bm = 512
    ni = npad // bm
    out = pl.pallas_call(
        _relayout_kernel,
        grid=(ni, nc),
        in_specs=[pl.BlockSpec((bm, 128), lambda i, j: (i, j))],
        out_specs=pl.BlockSpec((bm, 128), lambda i, j: (j * ni + i, 0)),
        out_shape=jax.ShapeDtypeStruct((nc * npad, 128), jnp.float32),
        compiler_params=pltpu.CompilerParams(
            dimension_semantics=("parallel", "parallel")),
        interpret=_INTERPRET,
    )(v)
    return out


def _ln_relu_kernel(x_ref, g_ref, b_ref, o_ref, *, d):
    x = x_ref[...]
    mu = jnp.sum(x, axis=-1, keepdims=True) / d
    xc = x - mu
    var = jnp.sum(jnp.where(lax.broadcasted_iota(jnp.int32, x.shape, 1) < d,
                            xc * xc, 0.0), axis=-1, keepdims=True) / d
    y = g_ref[...] * xc * lax.rsqrt(var + 1e-5) + b_ref[...]
    o_ref[...] = jnp.maximum(y, 0.0)


def _ln_relu(x, g, b, bm=256):
    R, D = x.shape
    d = g.shape[0]
    xp = _pad_to(_pad_to(x, 0, bm), 1, 128)
    Rp, Dp = xp.shape
    gp = _pad_to(g[None, :], 1, Dp)[:, :Dp]
    bp = _pad_to(b[None, :], 1, Dp)[:, :Dp]
    out = pl.pallas_call(
        functools.partial(_ln_relu_kernel, d=d),
        grid=(Rp // bm,),
        in_specs=[pl.BlockSpec((bm, Dp), lambda i: (i, 0)),
                  pl.BlockSpec((1, Dp), lambda i: (0, 0)),
                  pl.BlockSpec((1, Dp), lambda i: (0, 0))],
        out_specs=pl.BlockSpec((bm, Dp), lambda i: (i, 0)),
        out_shape=jax.ShapeDtypeStruct((Rp, Dp), jnp.float32),
        interpret=_INTERPRET,
    )(xp, gp, bp)
    return out[:R]


def _sigmoid_kernel(x_ref, o_ref):
    o_ref[...] = jax.nn.sigmoid(x_ref[...])


def _sigmoid(x, bm=512):
    R, D = x.shape
    xp = _pad_to(_pad_to(x, 0, bm), 1, 128)
    Rp, Dp = xp.shape
    out = pl.pallas_call(
        _sigmoid_kernel,
        grid=(Rp // bm,),
        in_specs=[pl.BlockSpec((bm, Dp), lambda i: (i, 0))],
        out_specs=pl.BlockSpec((bm, Dp), lambda i: (i, 0)),
        out_shape=jax.ShapeDtypeStruct((Rp, Dp), jnp.float32),
        interpret=_INTERPRET,
    )(xp)
    return out[:R, :D]


def _sc_mesh():
    return plsc.VectorSubcoreMesh(core_axis_name="c", subcore_axis_name="s")


def _hsum16(v):
    s = v[0]
    for l in range(1, _L):
        s = s + v[l]
    return s


def _hmax16(v):
    s = v[0]
    for l in range(1, _L):
        s = jnp.maximum(s, v[l])
    return s


def _vec16(scalars):
    lanes = lax.iota(jnp.int32, _L)
    v = jnp.full((_L,), scalars[0], jnp.float32)
    for l in range(1, _L):
        v = jnp.where(lanes == l, jnp.full((_L,), scalars[l], jnp.float32), v)
    return v


def _alpha_call(qp, kp, ep, srcp, dstp, scale):
    Npad, Dp = qp.shape
    Epad = ep.shape[0]
    EPW = Epad // _NW
    G = 8
    ng = EPW // G
    nv = Dp // _L

    @functools.partial(
        pl.kernel, mesh=_sc_mesh(),
        out_type=jax.ShapeDtypeStruct((Epad,), jnp.float32),
        scratch_types=[
            pltpu.VMEM((EPW,), jnp.int32),
            pltpu.VMEM((EPW,), jnp.int32),
            pltpu.VMEM((G, Dp), jnp.float32),
            pltpu.VMEM((G, Dp), jnp.float32),
            pltpu.VMEM((G, Dp), jnp.float32),
            pltpu.VMEM((EPW * _L,), jnp.float32),
            pltpu.VMEM((EPW,), jnp.float32),
            pltpu.SemaphoreType.DMA,
        ])
    def k_alpha(q_h, k_h, e_h, src_h, dst_h, al_h,
                src_v, dst_v, qb, kb, eb, pb, ab, sem):
        wid = lax.axis_index("s") * _NC + lax.axis_index("c")
        base = wid * EPW
        pltpu.sync_copy(src_h.at[pl.ds(base, EPW)], src_v)
        pltpu.sync_copy(dst_h.at[pl.ds(base, EPW)], dst_v)

        def group(g, carry):
            gb = g * G
            cq = pltpu.async_copy(q_h.at[dst_v.at[pl.ds(gb, G)]], qb, sem)
            ck = pltpu.async_copy(k_h.at[src_v.at[pl.ds(gb, G)]], kb, sem)
            ce = pltpu.async_copy(e_h.at[pl.ds(base + gb, G)], eb, sem)
            cq.wait()
            ck.wait()
            ce.wait()
            for i in range(G):
                def dot_step(j, acc):
                    sl = pl.ds(j * _L, _L)
                    return acc + qb[i, sl] * (kb[i, sl] + eb[i, sl])
                acc = lax.fori_loop(0, nv, dot_step,
                                    jnp.zeros((_L,), jnp.float32))
                pb[pl.ds((gb + i) * _L, _L)] = acc
            return carry

        lax.fori_loop(0, ng, group, 0)

        def batch(b, carry):
            sums = [_hsum16(pb[pl.ds((b * _L + l) * _L, _L)])
                    for l in range(_L)]
            ab[pl.ds(b * _L, _L)] = _vec16(sums) * scale
            return carry

        lax.fori_loop(0, EPW // _L, batch, 0)
        pltpu.sync_copy(ab, al_h.at[pl.ds(base, EPW)])

    return k_alpha(qp, kp, ep, srcp, dstp)


def _softmax_call(alpha, dstp, n_edges, npad):
    Epad = alpha.shape[0]
    EPT = Epad // _NS
    nch = EPT // _L
    slab = npad // _NS

    slabn = npad // _NS

    @functools.partial(
        pl.kernel, mesh=_sc_mesh(),
        out_type=jax.ShapeDtypeStruct((Epad,), jnp.float32),
        scratch_types=[
            pltpu.VMEM((EPT,), jnp.float32),
            pltpu.VMEM((EPT,), jnp.int32),
            pltpu.VMEM((EPT,), jnp.float32),
            pltpu.VMEM((EPT,), jnp.float32),
            pltpu.VMEM((npad,), jnp.float32),
            pltpu.VMEM((_L,), jnp.float32),
            pltpu.VMEM((_NS * _L,), jnp.float32),
            pltpu.VMEM((slabn,), jnp.float32),
            pltpu.VMEM((slabn,), jnp.float32),
            pltpu.VMEM_SHARED((_NS * _L,), jnp.float32),
            pltpu.VMEM_SHARED((_NS * npad,), jnp.float32),
            pltpu.VMEM_SHARED((npad,), jnp.float32),
            pltpu.SemaphoreType.DMA,
        ])
    def k_soft(a_h, dst_h, al_h,
               av, dv, exv, alv, sfl, mrow, mall, acc, tmp,
               mx_sh, s_all, s_comb, sem):
        cid = lax.axis_index("c")
        tid = lax.axis_index("s")

        @pl.when(cid == 0)
        def _():
            base = tid * EPT
            pltpu.sync_copy(a_h.at[pl.ds(base, EPT)], av)
            pltpu.sync_copy(dst_h.at[pl.ds(base, EPT)], dv)
            lanes = lax.iota(jnp.int32, _L)

            def mstep(i, m):
                idx = base + i * _L + lanes
                a = av[pl.ds(i * _L, _L)]
                return jnp.maximum(m, jnp.where(idx < n_edges, a, -1e30))

            m16 = lax.fori_loop(0, nch, mstep,
                                jnp.full((_L,), -1e30, jnp.float32))
            mrow[...] = jnp.full((_L,), _hmax16(m16), jnp.float32)
            pltpu.sync_copy(mrow, mx_sh.at[pl.ds(tid * _L, _L)])
            plsc.subcore_barrier()
            pltpu.sync_copy(mx_sh, mall)

            def mstep2(i, m):
                return jnp.maximum(m, mall[pl.ds(i * _L, _L)])

            mg = _hmax16(lax.fori_loop(0, _NS, mstep2,
                                       jnp.full((_L,), -1e30, jnp.float32)))

            def estep(i, c):
                sl = pl.ds(i * _L, _L)
                idx = base + i * _L + lanes
                exv[sl] = jnp.where(idx < n_edges,
                                    jnp.exp(av[sl] - mg), 0.0)
                return c

            lax.fori_loop(0, nch, estep, 0)

            def zstep(i, c):
                sfl[pl.ds(i * _L, _L)] = jnp.zeros((_L,), jnp.float32)
                return c

            lax.fori_loop(0, npad // _L, zstep, 0)

            def sstep(i, c):
                d16 = dv[pl.ds(i * _L, _L)]
                ex16 = exv[pl.ds(i * _L, _L)]
                for l in range(_L):
                    d = d16[l]
                    off = (d // _L) * _L
                    lp = d - off
                    cur = sfl[pl.ds(off, _L)]
                    upd = jnp.where(lanes == lp,
                                    jnp.full((_L,), ex16[l], jnp.float32),
                                    jnp.zeros((_L,), jnp.float32))
                    sfl[pl.ds(off, _L)] = cur + upd
                return c

            lax.fori_loop(0, nch, sstep, 0)

            pltpu.sync_copy(sfl, s_all.at[pl.ds(tid * npad, npad)])
            plsc.subcore_barrier()

            def zacc(i, c):
                acc[pl.ds(i * _L, _L)] = jnp.zeros((_L,), jnp.float32)
                return c

            lax.fori_loop(0, slabn // _L, zacc, 0)

            def jsum(j, c):
                pltpu.sync_copy(
                    s_all.at[pl.ds(j * npad + tid * slabn, slabn)], tmp)

                def astep(i, c2):
                    sl = pl.ds(i * _L, _L)
                    acc[sl] = acc[sl] + tmp[sl]
                    return c2

                lax.fori_loop(0, slabn // _L, astep, 0)
                return c

            lax.fori_loop(0, _NS, jsum, 0)
            pltpu.sync_copy(acc, s_comb.at[pl.ds(tid * slabn, slabn)])
            plsc.subcore_barrier()
            pltpu.sync_copy(s_comb, sfl)

            def nstep(i, c):
                sl = pl.ds(i * _L, _L)
                d16 = dv[sl]
                svals = []
                for l in range(_L):
                    d = d16[l]
                    off = (d // _L) * _L
                    lp = d - off
                    slot = sfl[pl.ds(off, _L)]
                    svals.append(_hsum16(jnp.where(
                        lanes == lp, slot, jnp.zeros((_L,), jnp.float32))))
                s16 = _vec16(svals)
                alv[sl] = exv[sl] / (s16 + 1e-16)
                return c

            lax.fori_loop(0, nch, nstep, 0)
            pltpu.sync_copy(alv, al_h.at[pl.ds(base, EPT)])

    return k_soft(alpha, dstp)


def _agg_call(vt2, ep, srcp, dstp, al, npad):
    Epad, Dp = ep.shape
    nc = Dp // 128
    EPT = Epad // _NS
    G = 64
    ng = EPT // G
    ncl = -(-nc // _NC)
    slab = npad // _NS

    @functools.partial(
        pl.kernel, mesh=_sc_mesh(),
        out_type=jax.ShapeDtypeStruct((npad, Dp), jnp.float32),
        scratch_types=[
            pltpu.VMEM((EPT,), jnp.int32),
            pltpu.VMEM((EPT,), jnp.int32),
            pltpu.VMEM((EPT,), jnp.float32),
            [pltpu.VMEM((G,), jnp.int32) for _ in range(2)],
            [pltpu.VMEM((G,), jnp.int32) for _ in range(2)],
            [pltpu.VMEM((G, 128), jnp.float32) for _ in range(2)],
            [pltpu.VMEM((G, 128), jnp.float32) for _ in range(2)],
            pltpu.VMEM((G, 128), jnp.float32),
            pltpu.VMEM((64, 128), jnp.float32),
            pltpu.VMEM_SHARED((npad, 128), jnp.float32),
            [pltpu.SemaphoreType.DMA for _ in range(2)],
        ])
    def k_agg(v_h, e_h, src_h, dst_h, al_h, out_h,
              src_v, dst_v, al_v, sg, dg, vb, eb, ob, zb, acc_sh, sem):
        cid = lax.axis_index("c")
        tid = lax.axis_index("s")
        base = tid * EPT
        pltpu.sync_copy(src_h.at[pl.ds(base, EPT)], src_v)
        pltpu.sync_copy(dst_h.at[pl.ds(base, EPT)], dst_v)
        pltpu.sync_copy(al_h.at[pl.ds(base, EPT)], al_v)

        def zrow(i, c):
            zb[i, :] = jnp.zeros((128,), jnp.float32)
            return c

        lax.fori_loop(0, 64, zrow, 0)

        def chunk(j, carry):
            ci = j * _NC + cid

            @pl.when(ci < nc)
            def _():
                def zsh(z, c):
                    pltpu.sync_copy(
                        zb, acc_sh.at[pl.ds(tid * slab + z * 64, 64)])
                    return c

                lax.fori_loop(0, slab // 64, zsh, 0)
                plsc.subcore_barrier()

                def fire(g, p):
                    gb = g * G
                    for t in range(G // _L):
                        sl_s = pl.ds(gb + t * _L, _L)
                        sl_d = pl.ds(t * _L, _L)
                        sg[p][sl_d] = src_v[sl_s] + ci * npad
                        dg[p][sl_d] = dst_v[sl_s]
                    pltpu.async_copy(v_h.at[sg[p]], vb[p], sem[p])
                    pltpu.async_copy(
                        e_h.at[pl.ds(base + gb, G), pl.ds(ci * 128, 128)],
                        eb[p], sem[p])

                def drain(p):
                    pltpu.make_async_copy(
                        v_h.at[sg[p]], vb[p], sem[p]).wait()
                    pltpu.make_async_copy(
                        e_h.at[pl.ds(0, G), pl.ds(0, 128)],
                        eb[p], sem[p]).wait()

                def work(g, p):
                    gb = g * G
                    for t in range(G // _L):
                        al16 = al_v[pl.ds(gb + t * _L, _L)]
                        for l in range(_L):
                            i = t * _L + l
                            a_sc = al16[l]
                            for j8 in range(128 // _L):
                                sl = pl.ds(j8 * _L, _L)
                                ob[i, sl] = (vb[p][i, sl]
                                             + eb[p][i, sl]) * a_sc
                    pltpu.sync_copy(ob, acc_sh.at[dg[p]], add=True)

                fire(0, 0)

                def gpair(q, c):
                    g0 = q * 2
                    fire(g0 + 1, 1)
                    drain(0)
                    work(g0, 0)

                    @pl.when(g0 + 2 < ng)
                    def _():
                        fire(g0 + 2, 0)
                    drain(1)
                    work(g0 + 1, 1)
                    return c

                lax.fori_loop(0, ng // 2, gpair, 0)
                plsc.subcore_barrier()
                pltpu.sync_copy(
                    acc_sh.at[pl.ds(tid * slab, slab)],
                    out_h.at[pl.ds(tid * slab, slab), pl.ds(ci * 128, 128)])

            return carry

        lax.fori_loop(0, ncl, chunk, 0)

    return k_agg(vt2, ep, srcp, dstp, al)


def _edge_phase_sc(qp, kp, vp, ep, srcp, dstp, n_edges, d_out):
    npad, dp = qp.shape
    nc = dp // 128
    scale = 1.0 / math.sqrt(float(d_out))
    alpha = _alpha_call(qp, kp, ep, srcp, dstp, scale)
    al = _softmax_call(alpha, dstp, n_edges, npad)
    vt2 = _chunk_rows(vp)
    agg = _agg_call(vt2, ep, srcp, dstp, al, npad)
    return agg, al


_NPAD = 5120
_EPAD = 10240


def _tconv(xp, eap, srcp, dstp, p, d_out, n_edges):
    dp = _ceil_to(d_out, 128)
    q = _mm_padded(xp, p['Wq'], dp, bm=1024)
    k = _mm_padded(xp, p['Wk'], dp, bm=1024)
    v = _mm_padded(xp, p['Wv'], dp, bm=1024)
    e = _mm_padded(eap, p['We'], dp, bm=1024)
    agg, al = _edge_phase_sc(q, k, v, e, srcp, dstp, n_edges, d_out)
    s = _mm_padded(xp, p['Ws'], dp, bm=1024)
    return agg + s, al


def kernel(x, edge_index, edge_attr, params):
    n, _ = x.shape
    e_cnt = edge_attr.shape[0]
    src = _pad_to(edge_index[0], 0, _EPAD)
    dst = _pad_to(edge_index[1], 0, _EPAD)
    xp = _pad_to(x, 0, _NPAD)
    ea0 = _pad_to(edge_attr, 0, _EPAD)
    ea = ea0
    dims = [3400, 2800, 2200, 1600, 1000]
    for i, d_out in enumerate(dims):
        p = params['conv%d' % (i + 1)]
        xp, al = _tconv(xp, ea, src, dst, p, d_out, e_cnt)
        if i < 4:
            nrm = params['norm%d' % (i + 1)]
            xp = _ln_relu(xp, nrm['g'], nrm['b'])[:, :_ceil_to(d_out, 128)]
            nrm1 = params['norm%d_1' % (i + 1)]
            ea = _ln_relu(jnp.concatenate([ea0, al[:, None]], axis=1),
                          nrm1['g'], nrm1['b'])[:, :24]
    return _sigmoid(xp[:n, :1000])


# speedup vs baseline: 1.2393x; 1.0240x over previous
import functools
import math

import jax
import jax.numpy as jnp
from jax import lax
from jax.experimental import pallas as pl
from jax.experimental.pallas import tpu as pltpu
from jax.experimental.pallas import tpu_sc as plsc

_INTERPRET = False

_NC = 2
_NS = 16
_L = 16
_NW = _NC * _NS


def _pad_to(a, axis, m):
    s = a.shape[axis]
    p = (-s) % m
    if p == 0:
        return a
    pads = [(0, 0)] * a.ndim
    pads[axis] = (0, p)
    return jnp.pad(a, pads)


def _ceil_to(x, m):
    return -(-x // m) * m


def _mm_kernel(a_ref, b_ref, o_ref, *, kk, nn, bk, bn, nk):
    k = pl.program_id(2)
    j = pl.program_id(1)

    @pl.when(k == 0)
    def _():
        o_ref[...] = jnp.zeros_like(o_ref)

    a = a_ref[...]
    kcol = lax.broadcasted_iota(jnp.int32, a.shape, 1) + k * bk
    a = jnp.where(kcol < kk, a, 0.0)
    o_ref[...] += lax.dot_general(
        a, b_ref[...], (((1,), (0,)), ((), ())),
        preferred_element_type=jnp.float32,
        precision=lax.Precision.DEFAULT)

    @pl.when(k == nk - 1)
    def _():
        ocol = lax.broadcasted_iota(jnp.int32, o_ref.shape, 1) + j * bn
        o_ref[...] = jnp.where(ocol < nn, o_ref[...], 0.0)


def _pick_bn(npad):
    m = npad // 128
    best = 1
    for nj in range(1, m + 1):
        if m % nj == 0 and (m // nj) * 128 <= 2048:
            best = m // nj
            break
    return best * 128


def _mm_padded(a, b, npad, bm=1024, bk=512):
    M, K = a.shape
    _, N = b.shape
    bk = min(bk, _ceil_to(K, 128))
    bn = _pick_bn(npad)
    nk = -(-K // bk)
    grid = (M // bm, npad // bn, nk)
    return pl.pallas_call(
        functools.partial(_mm_kernel, kk=K, nn=N, bk=bk, bn=bn, nk=nk),
        grid=grid,
        in_specs=[pl.BlockSpec((bm, bk), lambda i, j, k: (i, k)),
                  pl.BlockSpec((bk, bn), lambda i, j, k: (k, j))],
        out_specs=pl.BlockSpec((bm, bn), lambda i, j, k: (i, j)),
        out_shape=jax.ShapeDtypeStruct((M, npad), jnp.float32),
        compiler_params=pltpu.CompilerParams(
            dimension_semantics=("parallel", "parallel", "arbitrary")),
        interpret=_INTERPRET,
    )(a, b)


def _relayout_kernel(x_ref, o_ref):
    o_ref[...] = x_ref[...]


def _chunk_rows(v):
    npad, dp = v.shape
    nc = dp // 128
    bm = 512
    ni = npad // bm
    out = pl.pallas_call(
        _relayout_kernel,
        grid=(ni, nc),
        in_specs=[pl.BlockSpec((bm, 128), lambda i, j: (i, j))],
        out_specs=pl.BlockSpec((bm, 128), lambda i, j: (j * ni + i, 0)),
        out_shape=jax.ShapeDtypeStruct((nc * npad, 128), jnp.float32),
        compiler_params=pltpu.CompilerParams(
            dimension_semantics=("parallel", "parallel")),
        interpret=_INTERPRET,
    )(v)
    return out


def _ln_relu_kernel(x_ref, g_ref, b_ref, o_ref, *, d):
    x = x_ref[...]
    mu = jnp.sum(x, axis=-1, keepdims=True) / d
    xc = x - mu
    var = jnp.sum(jnp.where(lax.broadcasted_iota(jnp.int32, x.shape, 1) < d,
                            xc * xc, 0.0), axis=-1, keepdims=True) / d
    y = g_ref[...] * xc * lax.rsqrt(var + 1e-5) + b_ref[...]
    o_ref[...] = jnp.maximum(y, 0.0)


def _ln_relu(x, g, b, bm=256):
    R, D = x.shape
    d = g.shape[0]
    xp = _pad_to(_pad_to(x, 0, bm), 1, 128)
    Rp, Dp = xp.shape
    gp = _pad_to(g[None, :], 1, Dp)[:, :Dp]
    bp = _pad_to(b[None, :], 1, Dp)[:, :Dp]
    out = pl.pallas_call(
        functools.partial(_ln_relu_kernel, d=d),
        grid=(Rp // bm,),
        in_specs=[pl.BlockSpec((bm, Dp), lambda i: (i, 0)),
                  pl.BlockSpec((1, Dp), lambda i: (0, 0)),
                  pl.BlockSpec((1, Dp), lambda i: (0, 0))],
        out_specs=pl.BlockSpec((bm, Dp), lambda i: (i, 0)),
        out_shape=jax.ShapeDtypeStruct((Rp, Dp), jnp.float32),
        interpret=_INTERPRET,
    )(xp, gp, bp)
    return out[:R]


def _sigmoid_kernel(x_ref, o_ref):
    o_ref[...] = jax.nn.sigmoid(x_ref[...])


def _sigmoid(x, bm=512):
    R, D = x.shape
    xp = _pad_to(_pad_to(x, 0, bm), 1, 128)
    Rp, Dp = xp.shape
    out = pl.pallas_call(
        _sigmoid_kernel,
        grid=(Rp // bm,),
        in_specs=[pl.BlockSpec((bm, Dp), lambda i: (i, 0))],
        out_specs=pl.BlockSpec((bm, Dp), lambda i: (i, 0)),
        out_shape=jax.ShapeDtypeStruct((Rp, Dp), jnp.float32),
        interpret=_INTERPRET,
    )(xp)
    return out[:R, :D]


def _sc_mesh():
    return plsc.VectorSubcoreMesh(core_axis_name="c", subcore_axis_name="s")


def _hsum16(v):
    s = v[0]
    for l in range(1, _L):
        s = s + v[l]
    return s


def _hmax16(v):
    s = v[0]
    for l in range(1, _L):
        s = jnp.maximum(s, v[l])
    return s


def _vec16(scalars):
    lanes = lax.iota(jnp.int32, _L)
    v = jnp.full((_L,), scalars[0], jnp.float32)
    for l in range(1, _L):
        v = jnp.where(lanes == l, jnp.full((_L,), scalars[l], jnp.float32), v)
    return v


def _alpha_call(qp, kp, ep, src2, dst2, scale):
    Npad, Dp = qp.shape
    Epad = ep.shape[0]
    EPW = Epad // _NW
    G = 8
    ng = EPW // G
    H = Dp // 2
    nv = H // _L
    q2 = qp.reshape(2 * Npad, H)
    k2 = kp.reshape(2 * Npad, H)

    @functools.partial(
        pl.kernel, mesh=_sc_mesh(),
        out_type=jax.ShapeDtypeStruct((Epad,), jnp.float32),
        scratch_types=[
            [pltpu.VMEM((EPW,), jnp.int32) for _ in range(2)],
            [pltpu.VMEM((EPW,), jnp.int32) for _ in range(2)],
            [pltpu.VMEM((G, H), jnp.float32) for _ in range(2)],
            [pltpu.VMEM((G, H), jnp.float32) for _ in range(2)],
            [pltpu.VMEM((G, H), jnp.float32) for _ in range(2)],
            pltpu.VMEM((EPW * _L,), jnp.float32),
            pltpu.VMEM((EPW,), jnp.float32),
            [pltpu.SemaphoreType.DMA for _ in range(2)],
        ])
    def k_alpha(q_h, k_h, e_h, src_h, dst_h, al_h,
                src_v, dst_v, qb, kb, eb, pb, ab, sem):
        wid = lax.axis_index("s") * _NC + lax.axis_index("c")
        base = wid * EPW
        for h in range(2):
            pltpu.sync_copy(src_h.at[pl.ds(h * Epad + base, EPW)], src_v[h])
            pltpu.sync_copy(dst_h.at[pl.ds(h * Epad + base, EPW)], dst_v[h])

        def fire(g, h):
            gb = g * G
            pltpu.async_copy(
                q_h.at[dst_v[h].at[pl.ds(gb, G)]], qb[h], sem[h])
            pltpu.async_copy(
                k_h.at[src_v[h].at[pl.ds(gb, G)]], kb[h], sem[h])
            pltpu.async_copy(
                e_h.at[pl.ds(base + gb, G), pl.ds(h * H, H)], eb[h], sem[h])

        def drain(h):
            pltpu.make_async_copy(
                q_h.at[dst_v[h].at[pl.ds(0, G)]], qb[h], sem[h]).wait()
            pltpu.make_async_copy(
                k_h.at[src_v[h].at[pl.ds(0, G)]], kb[h], sem[h]).wait()
            pltpu.make_async_copy(
                e_h.at[pl.ds(0, G), pl.ds(0, H)], eb[h], sem[h]).wait()

        def compute(g, h, first):
            gb = g * G
            for i in range(G):
                def dot_step(j, acc):
                    sl = pl.ds(j * _L, _L)
                    return acc + qb[h][i, sl] * (kb[h][i, sl] + eb[h][i, sl])
                acc = lax.fori_loop(0, nv, dot_step,
                                    jnp.zeros((_L,), jnp.float32))
                po = pl.ds((gb + i) * _L, _L)
                if first:
                    pb[po] = acc
                else:
                    pb[po] = pb[po] + acc

        fire(0, 0)

        def group(g, carry):
            fire(g, 1)
            drain(0)
            compute(g, 0, True)

            @pl.when(g + 1 < ng)
            def _():
                fire(g + 1, 0)
            drain(1)
            compute(g, 1, False)
            return carry

        lax.fori_loop(0, ng, group, 0)

        def batch(b, carry):
            sums = [_hsum16(pb[pl.ds((b * _L + l) * _L, _L)])
                    for l in range(_L)]
            ab[pl.ds(b * _L, _L)] = _vec16(sums) * scale
            return carry

        lax.fori_loop(0, EPW // _L, batch, 0)
        pltpu.sync_copy(ab, al_h.at[pl.ds(base, EPW)])

    return k_alpha(q2, k2, ep, src2, dst2)


def _softmax_call(alpha, dstp, n_edges, npad):
    Epad = alpha.shape[0]
    EPT = Epad // _NS
    nch = EPT // _L
    slab = npad // _NS

    slabn = npad // _NS

    @functools.partial(
        pl.kernel, mesh=_sc_mesh(),
        out_type=jax.ShapeDtypeStruct((Epad,), jnp.float32),
        scratch_types=[
            pltpu.VMEM((EPT,), jnp.float32),
            pltpu.VMEM((EPT,), jnp.int32),
            pltpu.VMEM((EPT,), jnp.float32),
            pltpu.VMEM((EPT,), jnp.float32),
            pltpu.VMEM((npad,), jnp.float32),
            pltpu.VMEM((_L,), jnp.float32),
            pltpu.VMEM((_NS * _L,), jnp.float32),
            pltpu.VMEM((slabn,), jnp.float32),
            pltpu.VMEM((slabn,), jnp.float32),
            pltpu.VMEM_SHARED((_NS * _L,), jnp.float32),
            pltpu.VMEM_SHARED((_NS * npad,), jnp.float32),
            pltpu.VMEM_SHARED((npad,), jnp.float32),
            pltpu.SemaphoreType.DMA,
        ])
    def k_soft(a_h, dst_h, al_h,
               av, dv, exv, alv, sfl, mrow, mall, acc, tmp,
               mx_sh, s_all, s_comb, sem):
        cid = lax.axis_index("c")
        tid = lax.axis_index("s")

        @pl.when(cid == 0)
        def _():
            base = tid * EPT
            pltpu.sync_copy(a_h.at[pl.ds(base, EPT)], av)
            pltpu.sync_copy(dst_h.at[pl.ds(base, EPT)], dv)
            lanes = lax.iota(jnp.int32, _L)

            def mstep(i, m):
                idx = base + i * _L + lanes
                a = av[pl.ds(i * _L, _L)]
                return jnp.maximum(m, jnp.where(idx < n_edges, a, -1e30))

            m16 = lax.fori_loop(0, nch, mstep,
                                jnp.full((_L,), -1e30, jnp.float32))
            mrow[...] = jnp.full((_L,), _hmax16(m16), jnp.float32)
            pltpu.sync_copy(mrow, mx_sh.at[pl.ds(tid * _L, _L)])
            plsc.subcore_barrier()
            pltpu.sync_copy(mx_sh, mall)

            def mstep2(i, m):
                return jnp.maximum(m, mall[pl.ds(i * _L, _L)])

            mg = _hmax16(lax.fori_loop(0, _NS, mstep2,
                                       jnp.full((_L,), -1e30, jnp.float32)))

            def estep(i, c):
                sl = pl.ds(i * _L, _L)
                idx = base + i * _L + lanes
                exv[sl] = jnp.where(idx < n_edges,
                                    jnp.exp(av[sl] - mg), 0.0)
                return c

            lax.fori_loop(0, nch, estep, 0)

            def zstep(i, c):
                sfl[pl.ds(i * _L, _L)] = jnp.zeros((_L,), jnp.float32)
                return c

            lax.fori_loop(0, npad // _L, zstep, 0)

            def sstep(i, c):
                d16 = dv[pl.ds(i * _L, _L)]
                ex16 = exv[pl.ds(i * _L, _L)]
                for l in range(_L):
                    d = d16[l]
                    off = (d // _L) * _L
                    lp = d - off
                    cur = sfl[pl.ds(off, _L)]
                    upd = jnp.where(lanes == lp,
                                    jnp.full((_L,), ex16[l], jnp.float32),
                                    jnp.zeros((_L,), jnp.float32))
                    sfl[pl.ds(off, _L)] = cur + upd
                return c

            lax.fori_loop(0, nch, sstep, 0)

            pltpu.sync_copy(sfl, s_all.at[pl.ds(tid * npad, npad)])
            plsc.subcore_barrier()

            def zacc(i, c):
                acc[pl.ds(i * _L, _L)] = jnp.zeros((_L,), jnp.float32)
                return c

            lax.fori_loop(0, slabn // _L, zacc, 0)

            def jsum(j, c):
                pltpu.sync_copy(
                    s_all.at[pl.ds(j * npad + tid * slabn, slabn)], tmp)

                def astep(i, c2):
                    sl = pl.ds(i * _L, _L)
                    acc[sl] = acc[sl] + tmp[sl]
                    return c2

                lax.fori_loop(0, slabn // _L, astep, 0)
                return c

            lax.fori_loop(0, _NS, jsum, 0)
            pltpu.sync_copy(acc, s_comb.at[pl.ds(tid * slabn, slabn)])
            plsc.subcore_barrier()
            pltpu.sync_copy(s_comb, sfl)

            def nstep(i, c):
                sl = pl.ds(i * _L, _L)
                d16 = dv[sl]
                svals = []
                for l in range(_L):
                    d = d16[l]
                    off = (d // _L) * _L
                    lp = d - off
                    slot = sfl[pl.ds(off, _L)]
                    svals.append(_hsum16(jnp.where(
                        lanes == lp, slot, jnp.zeros((_L,), jnp.float32))))
                s16 = _vec16(svals)
                alv[sl] = exv[sl] / (s16 + 1e-16)
                return c

            lax.fori_loop(0, nch, nstep, 0)
            pltpu.sync_copy(alv, al_h.at[pl.ds(base, EPT)])

    return k_soft(alpha, dstp)


def _agg_call(vt2, ep, srcp, dstp, al, npad):
    Epad, Dp = ep.shape
    nc = Dp // 128
    EPT = Epad // _NS
    G = 64
    ng = EPT // G
    ncl = -(-nc // _NC)
    slab = npad // _NS

    @functools.partial(
        pl.kernel, mesh=_sc_mesh(),
        out_type=jax.ShapeDtypeStruct((npad, Dp), jnp.float32),
        scratch_types=[
            pltpu.VMEM((EPT,), jnp.int32),
            pltpu.VMEM((EPT,), jnp.int32),
            pltpu.VMEM((EPT,), jnp.float32),
            [pltpu.VMEM((G,), jnp.int32) for _ in range(2)],
            [pltpu.VMEM((G,), jnp.int32) for _ in range(2)],
            [pltpu.VMEM((G, 128), jnp.float32) for _ in range(2)],
            [pltpu.VMEM((G, 128), jnp.float32) for _ in range(2)],
            pltpu.VMEM((G, 128), jnp.float32),
            pltpu.VMEM((64, 128), jnp.float32),
            pltpu.VMEM_SHARED((npad, 128), jnp.float32),
            [pltpu.SemaphoreType.DMA for _ in range(2)],
        ])
    def k_agg(v_h, e_h, src_h, dst_h, al_h, out_h,
              src_v, dst_v, al_v, sg, dg, vb, eb, ob, zb, acc_sh, sem):
        cid = lax.axis_index("c")
        tid = lax.axis_index("s")
        base = tid * EPT
        pltpu.sync_copy(src_h.at[pl.ds(base, EPT)], src_v)
        pltpu.sync_copy(dst_h.at[pl.ds(base, EPT)], dst_v)
        pltpu.sync_copy(al_h.at[pl.ds(base, EPT)], al_v)

        def zrow(i, c):
            zb[i, :] = jnp.zeros((128,), jnp.float32)
            return c

        lax.fori_loop(0, 64, zrow, 0)

        def chunk(j, carry):
            ci = j * _NC + cid

            @pl.when(ci < nc)
            def _():
                def zsh(z, c):
                    pltpu.sync_copy(
                        zb, acc_sh.at[pl.ds(tid * slab + z * 64, 64)])
                    return c

                lax.fori_loop(0, slab // 64, zsh, 0)
                plsc.subcore_barrier()

                def fire(g, p):
                    gb = g * G
                    for t in range(G // _L):
                        sl_s = pl.ds(gb + t * _L, _L)
                        sl_d = pl.ds(t * _L, _L)
                        sg[p][sl_d] = src_v[sl_s] + ci * npad
                        dg[p][sl_d] = dst_v[sl_s]
                    pltpu.async_copy(v_h.at[sg[p]], vb[p], sem[p])
                    pltpu.async_copy(
                        e_h.at[pl.ds(base + gb, G), pl.ds(ci * 128, 128)],
                        eb[p], sem[p])

                def drain(p):
                    pltpu.make_async_copy(
                        v_h.at[sg[p]], vb[p], sem[p]).wait()
                    pltpu.make_async_copy(
                        e_h.at[pl.ds(0, G), pl.ds(0, 128)],
                        eb[p], sem[p]).wait()

                def work(g, p):
                    gb = g * G
                    for t in range(G // _L):
                        al16 = al_v[pl.ds(gb + t * _L, _L)]
                        for l in range(_L):
                            i = t * _L + l
                            a_sc = al16[l]
                            for j8 in range(128 // _L):
                                sl = pl.ds(j8 * _L, _L)
                                ob[i, sl] = (vb[p][i, sl]
                                             + eb[p][i, sl]) * a_sc
                    pltpu.sync_copy(ob, acc_sh.at[dg[p]], add=True)

                fire(0, 0)

                def gpair(q, c):
                    g0 = q * 2
                    fire(g0 + 1, 1)
                    drain(0)
                    work(g0, 0)

                    @pl.when(g0 + 2 < ng)
                    def _():
                        fire(g0 + 2, 0)
                    drain(1)
                    work(g0 + 1, 1)
                    return c

                lax.fori_loop(0, ng // 2, gpair, 0)
                plsc.subcore_barrier()
                pltpu.sync_copy(
                    acc_sh.at[pl.ds(tid * slab, slab)],
                    out_h.at[pl.ds(tid * slab, slab), pl.ds(ci * 128, 128)])

            return carry

        lax.fori_loop(0, ncl, chunk, 0)

    return k_agg(vt2, ep, srcp, dstp, al)


def _edge_phase_sc(qp, kp, vp, ep, srcp, dstp, src2, dst2, n_edges, d_out):
    npad, dp = qp.shape
    nc = dp // 128
    scale = 1.0 / math.sqrt(float(d_out))
    alpha = _alpha_call(qp, kp, ep, src2, dst2, scale)
    al = _softmax_call(alpha, dstp, n_edges, npad)
    vt2 = _chunk_rows(vp)
    agg = _agg_call(vt2, ep, srcp, dstp, al, npad)
    return agg, al


_NPAD = 5120
_EPAD = 10240


def _tconv(xp, eap, srcp, dstp, src2, dst2, p, d_out, n_edges):
    dp = _ceil_to(d_out, 256)
    q = _mm_padded(xp, p['Wq'], dp, bm=1024)
    k = _mm_padded(xp, p['Wk'], dp, bm=1024)
    v = _mm_padded(xp, p['Wv'], dp, bm=1024)
    e = _mm_padded(eap, p['We'], dp, bm=1024)
    agg, al = _edge_phase_sc(q, k, v, e, srcp, dstp, src2, dst2,
                             n_edges, d_out)
    s = _mm_padded(xp, p['Ws'], dp, bm=1024)
    return agg + s, al


def kernel(x, edge_index, edge_attr, params):
    n, _ = x.shape
    e_cnt = edge_attr.shape[0]
    src = _pad_to(edge_index[0], 0, _EPAD)
    dst = _pad_to(edge_index[1], 0, _EPAD)
    src2 = jnp.concatenate([2 * src, 2 * src + 1])
    dst2 = jnp.concatenate([2 * dst, 2 * dst + 1])
    xp = _pad_to(x, 0, _NPAD)
    ea0 = _pad_to(edge_attr, 0, _EPAD)
    ea = ea0
    dims = [3400, 2800, 2200, 1600, 1000]
    for i, d_out in enumerate(dims):
        p = params['conv%d' % (i + 1)]
        xp, al = _tconv(xp, ea, src, dst, src2, dst2, p, d_out, e_cnt)
        if i < 4:
            nrm = params['norm%d' % (i + 1)]
            xp = _ln_relu(xp, nrm['g'], nrm['b'])
            nrm1 = params['norm%d_1' % (i + 1)]
            ea = _ln_relu(jnp.concatenate([ea0, al[:, None]], axis=1),
                          nrm1['g'], nrm1['b'])[:, :24]
    return _sigmoid(xp[:n, :1000])
